# baseline (speedup 1.0000x reference)
import functools

import jax
import jax.numpy as jnp
from jax import lax
from jax.experimental import pallas as pl
from jax.experimental.pallas import tpu as pltpu
from jax.experimental.pallas import tpu_sc as plsc

V, E, B, K = 1000000, 64, 16384, 20
NC, NS = 2, 16
NW = NC * NS
S = B // NW
C = 16
NCH = S // C
NIR = C * K // 64
NROW = S * K // 64
EV = E // 16


TBL = 384
NFULL = V // TBL
TOUT = TBL // 2
NIT = 82


def _sc_transpose(embcT, emboT):
  mesh = plsc.VectorSubcoreMesh(core_axis_name="c", subcore_axis_name="s")

  @functools.partial(
      pl.kernel, mesh=mesh,
      out_type=(jax.ShapeDtypeStruct((V // 2, 128), jnp.float32),
                jax.ShapeDtypeStruct((V // 2, 128), jnp.float32)),
      compiler_params=pltpu.CompilerParams(needs_layout_passes=False),
      scratch_types=[
          pltpu.VMEM((E, TBL), jnp.float32),
          pltpu.VMEM((E, TBL), jnp.float32),
          pltpu.VMEM((TOUT, 128), jnp.float32),
          pltpu.VMEM((TOUT, 128), jnp.float32),
          pltpu.VMEM((E, 64), jnp.float32),
          pltpu.VMEM((32, 128), jnp.float32),
          pltpu.SemaphoreType.DMA,
          pltpu.SemaphoreType.DMA,
          pltpu.SemaphoreType.DMA,
          pltpu.SemaphoreType.DMA,
      ])
  def k(embcT_hbm, emboT_hbm, outc_hbm, outo_hbm,
        fbuf0, fbuf1, obuf0, obuf1, tfbuf, tobuf, isem0, isem1, osem0, osem1):
    fbuf = (fbuf0, fbuf1)
    obuf = (obuf0, obuf1)
    isems = (isem0, isem1)
    osems = (osem0, osem1)
    wid = lax.axis_index("s") * NC + lax.axis_index("c")
    iota = jnp.arange(16, dtype=jnp.int32)
    evec = [iota + 16 * j for j in range(EV)]
    rvecs = [iota // 2 + wi * 8 for wi in range(TBL // 16)]
    cpar = (iota & 1) * 64

    def do_table(src, dst):
      def blk_of(it):
        return it * 32 + wid

      def issue(it, p):
        @pl.when(blk_of(it) < NFULL)
        def _():
          pltpu.async_copy(src.at[:, pl.ds(blk_of(it) * TBL, TBL)], fbuf[p],
                           isems[p])

      def wait_in(it, p):
        @pl.when(blk_of(it) < NFULL)
        def _():
          pltpu.make_async_copy(src.at[:, pl.ds(0, TBL)], fbuf[p],
                                isems[p]).wait()

      def owait(p):
        pltpu.make_async_copy(dst.at[pl.ds(0, TOUT), :], obuf[p],
                              osems[p]).wait()

      def compute(it, p):
        @pl.when(blk_of(it) < NFULL)
        def _():
          def feat(e, carry):
            cvec = cpar + e
            for wi in range(TBL // 16):
              v = fbuf[p][e, pl.ds(16 * wi, 16)]
              plsc.store_scatter(obuf[p], [rvecs[wi], cvec], v)
            return carry

          lax.fori_loop(0, E, feat, 0)
          pltpu.async_copy(obuf[p], dst.at[pl.ds(blk_of(it) * TOUT, TOUT), :],
                           osems[p])

      issue(0, 0)

      def outer(itp, carry):
        for lane in range(2):
          it = itp * 2 + lane
          issue(it + 1, (lane + 1) % 2)
          wait_in(it, lane)

          @pl.when(itp >= 1)
          def _():
            owait(lane)

          compute(it, lane)
        return carry

      lax.fori_loop(0, NIT // 2, outer, 0)
      owait(0)

      @pl.when(wid < (NFULL - (NIT - 1) * 32))
      def _():
        owait(1)

      @pl.when(wid == 31)
      def _():
        pltpu.sync_copy(src.at[:, pl.ds(NFULL * TBL, 64)], tfbuf)

        def trow(r, carry):
          w0 = 2 * r
          for j in range(EV):
            lo = plsc.load_gather(
                tfbuf, [evec[j], jnp.full((16,), w0, jnp.int32)])
            hi = plsc.load_gather(
                tfbuf, [evec[j], jnp.full((16,), w0 + 1, jnp.int32)])
            tobuf[r, pl.ds(16 * j, 16)] = lo
            tobuf[r, pl.ds(64 + 16 * j, 16)] = hi
          return carry

        lax.fori_loop(0, 32, trow, 0)
        pltpu.sync_copy(tobuf, dst.at[pl.ds(NFULL * TOUT, 32), :])

    do_table(embcT_hbm, outc_hbm)
    do_table(emboT_hbm, outo_hbm)

  return k(embcT, emboT)


def _sc_partials(center, outside, neg2d, embc2, embo2):
  mesh = plsc.VectorSubcoreMesh(core_axis_name="c", subcore_axis_name="s")

  @functools.partial(
      pl.kernel, mesh=mesh,
      out_type=jax.ShapeDtypeStruct((B, 32), jnp.float32),
      compiler_params=pltpu.CompilerParams(use_tc_tiling_on_sc=True),
      scratch_types=[
          pltpu.VMEM((S,), jnp.int32),
          pltpu.VMEM((S + 16,), jnp.int32),
          pltpu.VMEM((S,), jnp.int32),
          pltpu.VMEM((S + 16,), jnp.int32),
          pltpu.VMEM((NROW, 64), jnp.int32),
          pltpu.VMEM((S * K + 16,), jnp.int32),
          pltpu.VMEM((C, 2 * E), jnp.float32),
          pltpu.VMEM((C, 2 * E), jnp.float32),
          pltpu.VMEM((C, 2 * E), jnp.float32),
          pltpu.VMEM((C, 2 * E), jnp.float32),
          pltpu.VMEM((C * K, 2 * E), jnp.float32),
          pltpu.VMEM((C * K, 2 * E), jnp.float32),
          pltpu.VMEM((C, 32), jnp.float32),
          pltpu.VMEM((C, 32), jnp.float32),
          pltpu.SemaphoreType.DMA,
          pltpu.SemaphoreType.DMA,
          pltpu.SemaphoreType.DMA,
          pltpu.SemaphoreType.DMA,
      ])
  def k(center_hbm, outside_hbm, neg_hbm, embc_hbm, embo_hbm, out_hbm,
        cphy, coff, ophy, ooff, nphy, noff, crows0, crows1, orows0, orows1,
        nrows0, nrows1, outv0, outv1, sem0, sem1, osem0, osem1):
    crows = (crows0, crows1)
    orows = (orows0, orows1)
    nrows = (nrows0, nrows1)
    outv = (outv0, outv1)
    sems = (sem0, sem1)
    osems = (osem0, osem1)
    wid = lax.axis_index("s") * NC + lax.axis_index("c")
    base = wid * S

    pltpu.sync_copy(center_hbm.at[pl.ds(base, S)], cphy)
    pltpu.sync_copy(outside_hbm.at[pl.ds(base, S)], ophy)
    pltpu.sync_copy(neg_hbm.at[pl.ds(wid * NROW, NROW), :], nphy)

    def split_1d(ref, off_ref, t, _):
      v = ref[pl.ds(t * 16, 16)]
      off_ref[pl.ds(t * 16, 16)] = (v & 1) << 6
      ref[pl.ds(t * 16, 16)] = v >> 1
      return _

    lax.fori_loop(0, S // 16, functools.partial(split_1d, cphy, coff), 0)
    lax.fori_loop(0, S // 16, functools.partial(split_1d, ophy, ooff), 0)

    def split_neg(t, _):
      r = t // 4
      j = t % 4
      v = nphy[r, pl.ds(j * 16, 16)]
      noff[pl.ds(t * 16, 16)] = (v & 1) << 6
      nphy[r, pl.ds(j * 16, 16)] = v >> 1
      return _

    lax.fori_loop(0, NROW * 4, split_neg, 0)

    def issue(g, p):
      pltpu.async_copy(embc_hbm.at[cphy.at[pl.ds(g * C, C)]], crows[p],
                       sems[p])
      pltpu.async_copy(embo_hbm.at[ophy.at[pl.ds(g * C, C)]], orows[p],
                       sems[p])
      for j in range(NIR):
        pltpu.async_copy(embo_hbm.at[nphy.at[g * NIR + j]],
                         nrows[p].at[pl.ds(j * 64, 64)], sems[p])

    def wait(p):
      pltpu.make_async_copy(embc_hbm.at[pl.ds(0, C)], crows[p],
                            sems[p]).wait()
      pltpu.make_async_copy(embc_hbm.at[pl.ds(0, C)], orows[p],
                            sems[p]).wait()
      for j in range(NIR):
        pltpu.make_async_copy(embc_hbm.at[pl.ds(0, 64)],
                              nrows[p].at[pl.ds(j * 64, 64)],
                              sems[p]).wait()

    def compute(g, p):
      cr, orr, nr, ov = crows[p], orows[p], nrows[p], outv[p]

      def body2(lb, carry):
        bg = g * C + lb
        co = coff[pl.ds(bg, 16)][0]
        oo = ooff[pl.ds(bg, 16)][0]
        nov0 = noff[pl.ds(bg * K, 16)]
        nov1 = noff[pl.ds(bg * K + 16, 16)]
        nos = [nov0[kk] for kk in range(16)] + [nov1[kk] for kk in range(4)]
        cs = [cr[lb, pl.ds(co + 16 * j, 16)] for j in range(EV)]
        acc_o = cs[0] * orr[lb, pl.ds(oo, 16)]
        for j in range(1, EV):
          acc_o = acc_o + cs[j] * orr[lb, pl.ds(oo + 16 * j, 16)]
        acc_n = None
        for j in range(EV):
          s = None
          for kk in range(K):
            r = nr[lb * K + kk, pl.ds(nos[kk] + 16 * j, 16)]
            s = r if s is None else s + r
          t = s * cs[j]
          acc_n = t if acc_n is None else acc_n + t
        ov[lb, pl.ds(0, 16)] = acc_o
        ov[lb, pl.ds(16, 16)] = acc_n
        return carry

      lax.fori_loop(0, C, body2, 0)

    def flush(g, p):
      pltpu.async_copy(outv[p], out_hbm.at[pl.ds(base + g * C, C), :],
                       osems[p])

    def owait(p):
      pltpu.make_async_copy(out_hbm.at[pl.ds(0, C), :], outv[p],
                            osems[p]).wait()

    issue(0, 0)

    def outer(gp, carry):
      for lane in range(2):
        g = gp * 2 + lane

        @pl.when(g + 1 < NCH)
        def _():
          issue(g + 1, (lane + 1) % 2)

        wait(lane)

        @pl.when(g >= 2)
        def _():
          owait(lane)

        compute(g, lane)
        flush(g, lane)
      return carry

    lax.fori_loop(0, NCH // 2, outer, 0)
    owait(0)
    owait(1)

  return k(center, outside, neg2d, embc2, embo2)


def _logsig(x):
  return jnp.minimum(x, 0.0) - jnp.log1p(jnp.exp(-jnp.abs(x)))


def _finish_body(p_ref, o_ref):
  x = p_ref[...]
  uovc = jnp.sum(x[:, 0:16], axis=1)
  nd = jnp.sum(x[:, 16:32], axis=1)
  loss = _logsig(uovc) + _logsig(-nd)
  o_ref[...] = jnp.broadcast_to(-jnp.mean(loss), (1, 1))


def kernel(center, outside, negative, emb_center, emb_outside):
  c = center.reshape(B).astype(jnp.int32)
  o = outside.reshape(B).astype(jnp.int32)
  n = negative.reshape(B * K // 64, 64).astype(jnp.int32)
  embc2, embo2 = _sc_transpose(jnp.transpose(emb_center),
                               jnp.transpose(emb_outside))
  parts = _sc_partials(c, o, n, embc2, embo2)
  out = pl.pallas_call(
      _finish_body,
      out_shape=jax.ShapeDtypeStruct((1, 1), jnp.float32))(parts)
  return out[0, 0]


# speedup vs baseline: 2.2182x; 2.2182x over previous
import functools

import jax
import jax.numpy as jnp
from jax import lax
from jax.experimental import pallas as pl
from jax.experimental.pallas import tpu as pltpu
from jax.experimental.pallas import tpu_sc as plsc

V, E, B, K = 1000000, 64, 16384, 20
NC, NS = 2, 16
NW = NC * NS
S = B // NW
C = 32
NCH = S // C
NIR = C * K // 128
NROW = S * K // 128
EV = E // 16


def _sc_partials(center, outside, neg2d, emb_center, emb_outside):
  mesh = plsc.VectorSubcoreMesh(core_axis_name="c", subcore_axis_name="s")

  @functools.partial(
      pl.kernel, mesh=mesh,
      out_type=jax.ShapeDtypeStruct((B, 32), jnp.float32),
      compiler_params=pltpu.CompilerParams(use_tc_tiling_on_sc=False),
      scratch_types=[
          pltpu.VMEM((S,), jnp.int32),
          pltpu.VMEM((S,), jnp.int32),
          pltpu.VMEM((NROW, 128), jnp.int32),
          pltpu.VMEM((C, E), jnp.float32),
          pltpu.VMEM((C, E), jnp.float32),
          pltpu.VMEM((C, E), jnp.float32),
          pltpu.VMEM((C, E), jnp.float32),
          pltpu.VMEM((C * K, E), jnp.float32),
          pltpu.VMEM((C * K, E), jnp.float32),
          pltpu.VMEM((S, 32), jnp.float32),
          pltpu.SemaphoreType.DMA,
          pltpu.SemaphoreType.DMA,
      ])
  def k(center_hbm, outside_hbm, neg_hbm, embc_hbm, embo_hbm, out_hbm,
        cidx, oidx, nidx, crows0, crows1, orows0, orows1, nrows0, nrows1,
        outv, sem0, sem1):
    crows = (crows0, crows1)
    orows = (orows0, orows1)
    nrows = (nrows0, nrows1)
    sems = (sem0, sem1)
    wid = lax.axis_index("s") * NC + lax.axis_index("c")
    base = wid * S

    pltpu.sync_copy(center_hbm.at[pl.ds(base, S)], cidx)
    pltpu.sync_copy(outside_hbm.at[pl.ds(base, S)], oidx)
    pltpu.sync_copy(neg_hbm.at[pl.ds(wid * NROW, NROW), :], nidx)

    def issue(g, p):
      pltpu.async_copy(embc_hbm.at[cidx.at[pl.ds(g * C, C)]], crows[p], sems[p])
      pltpu.async_copy(embo_hbm.at[oidx.at[pl.ds(g * C, C)]], orows[p], sems[p])
      for j in range(NIR):
        pltpu.async_copy(embo_hbm.at[nidx.at[g * NIR + j]],
                         nrows[p].at[pl.ds(j * 128, 128)], sems[p])

    def wait(p):
      pltpu.make_async_copy(embc_hbm.at[pl.ds(0, C)], crows[p], sems[p]).wait()
      pltpu.make_async_copy(embo_hbm.at[pl.ds(0, C)], orows[p], sems[p]).wait()
      pltpu.make_async_copy(embo_hbm.at[pl.ds(0, C * K)], nrows[p],
                            sems[p]).wait()

    def compute(g, p):
      cr, orr, nr = crows[p], orows[p], nrows[p]

      def body(b, carry):
        cs = [cr[b, pl.ds(16 * j, 16)] for j in range(EV)]
        acc_o = cs[0] * orr[b, pl.ds(0, 16)]
        for j in range(1, EV):
          acc_o = acc_o + cs[j] * orr[b, pl.ds(16 * j, 16)]
        acc_n = None
        for j in range(EV):
          s = nr[b * K, pl.ds(16 * j, 16)]
          for kk in range(1, K):
            s = s + nr[b * K + kk, pl.ds(16 * j, 16)]
          t = s * cs[j]
          acc_n = t if acc_n is None else acc_n + t
        row = g * C + b
        outv[row, pl.ds(0, 16)] = acc_o
        outv[row, pl.ds(16, 16)] = acc_n
        return carry

      lax.fori_loop(0, C, body, 0)

    issue(0, 0)

    def outer(gp, carry):
      for lane in range(2):
        g = gp * 2 + lane

        @pl.when(g + 1 < NCH)
        def _():
          issue(g + 1, (lane + 1) % 2)

        wait(lane)
        compute(g, lane)
      return carry

    lax.fori_loop(0, NCH // 2, outer, 0)
    pltpu.sync_copy(outv, out_hbm.at[pl.ds(base, S), :])

  return k(center, outside, neg2d, emb_center, emb_outside)


def _logsig(x):
  return jnp.minimum(x, 0.0) - jnp.log1p(jnp.exp(-jnp.abs(x)))


def _finish_body(p_ref, o_ref):
  x = p_ref[...]
  uovc = jnp.sum(x[:, 0:16], axis=1)
  nd = jnp.sum(x[:, 16:32], axis=1)
  loss = _logsig(uovc) + _logsig(-nd)
  o_ref[...] = jnp.broadcast_to(-jnp.mean(loss), (1, 1))


def kernel(center, outside, negative, emb_center, emb_outside):
  c = center.reshape(B).astype(jnp.int32)
  o = outside.reshape(B).astype(jnp.int32)
  n = negative.reshape(B * K // 128, 128).astype(jnp.int32)
  parts = _sc_partials(c, o, n, emb_center, emb_outside)
  out = pl.pallas_call(
      _finish_body,
      out_shape=jax.ShapeDtypeStruct((1, 1), jnp.float32))(parts)
  return out[0, 0]
